# baseline (speedup 1.0000x reference)
import jax
import jax.numpy as jnp
from jax.experimental import pallas as pl

_NUM_LAYERS = 5
_G = 128


def _gin_layer(x, src, dst, W1, b1, g, be, W2, b2):
    agg = jax.ops.segment_sum(jnp.take(x, src, axis=0), dst, num_segments=x.shape[0])
    h = x + agg
    h = h @ W1 + b1
    mean = jnp.mean(h, axis=0)
    var = jnp.var(h, axis=0)
    h = (h - mean) / jnp.sqrt(var + 1e-5) * g + be
    h = jax.nn.relu(h)
    h = h @ W2 + b2
    return h


def _matvec_kernel(p_ref, w_ref, b_ref, o_ref):
    o_ref[:, :] = jnp.dot(p_ref[:, :], w_ref[:, :]) + b_ref[0, 0]


def kernel(x, edge_index, batch, W1_0, b1_0, g_0, be_0, W2_0, b2_0,
           W1_r, b1_r, g_r, be_r, W2_r, b2_r, Wc, bc):
    src = edge_index[0]
    dst = edge_index[1]
    h = _gin_layer(x, src, dst, W1_0, b1_0, g_0, be_0, W2_0, b2_0)
    h = jax.nn.relu(h)
    for i in range(_NUM_LAYERS - 1):
        h = _gin_layer(h, src, dst, W1_r[i], b1_r[i], g_r[i], be_r[i], W2_r[i], b2_r[i])
        h = jax.nn.relu(h)
    pooled = jax.ops.segment_sum(h, batch, num_segments=_G)
    out = pl.pallas_call(
        _matvec_kernel,
        out_shape=jax.ShapeDtypeStruct((_G, 1), jnp.float32),
    )(pooled, Wc, bc.reshape(1, 1))
    return out.reshape(-1)


# speedup vs baseline: 16.0070x; 16.0070x over previous
import functools

import jax
import jax.numpy as jnp
from jax import lax
from jax.experimental import pallas as pl
from jax.experimental.pallas import tpu as pltpu
from jax.experimental.pallas import tpu_sc as plsc

_N = 100000
_E = 1600000
_H = 32
_HH = _H // 2
_G = 128
_L = 5
_NC = 2
_NS = 16
_EDGE_PER_TILE = _E // _NS
_CH = 800
_NCHUNK = _EDGE_PER_TILE // _CH

_ZROW = 800
_ZCHUNKS = _N // _ZROW
_ZROUNDS = -(-_ZCHUNKS // _NS)

_POOL_CH = 1000
_HALF = _N // _NC
_POOL_CHUNKS_TOTAL = _HALF // _POOL_CH
_POOL_ROUNDS = -(-_POOL_CHUNKS_TOTAL // _NS)

_mesh = plsc.VectorSubcoreMesh(core_axis_name="c", subcore_axis_name="s")
_sc_params = pltpu.CompilerParams(use_tc_tiling_on_sc=False)


@functools.partial(
    pl.kernel,
    out_type=jax.ShapeDtypeStruct((_NC * _N, _HH), jnp.float32),
    mesh=_mesh,
    compiler_params=_sc_params,
    scratch_types=[
        pltpu.VMEM_SHARED((_N, _HH), jnp.float32),
        pltpu.VMEM((_CH,), jnp.int32),
        pltpu.VMEM((_CH,), jnp.int32),
        pltpu.VMEM((_CH,), jnp.int32),
        pltpu.VMEM((_CH,), jnp.int32),
        pltpu.VMEM((_CH, _HH), jnp.float32),
        pltpu.VMEM((_CH, _HH), jnp.float32),
        pltpu.SemaphoreType.DMA,
        pltpu.SemaphoreType.DMA,
        pltpu.SemaphoreType.DMA,
        pltpu.SemaphoreType.DMA,
    ],
)
def _sc_agg(x2_hbm, srcm_hbm, dst_hbm, out_hbm, acc, idx_s0, idx_s1, idx_d0,
            idx_d1, rows0, rows1, si0, si1, sg0, sg1):
    c = lax.axis_index("c")
    s = lax.axis_index("s")

    def zrow(i, carry):
        rows0[i] = jnp.zeros((16,), jnp.float32)
        return carry

    lax.fori_loop(0, _CH, zrow, 0)
    for j in range(_ZROUNDS):
        k = s + j * _NS

        @pl.when(k < _ZCHUNKS)
        def _():
            pltpu.sync_copy(rows0, acc.at[pl.ds(k * _ZROW, _ZROW)])

    plsc.subcore_barrier()
    base0 = s * _EDGE_PER_TILE
    idx_s = (idx_s0, idx_s1)
    idx_d = (idx_d0, idx_d1)
    rows = (rows0, rows1)
    si = (si0, si1)
    sg = (sg0, sg1)

    def load_idx(k, b):
        base = base0 + k * _CH
        pltpu.async_copy(srcm_hbm.at[pl.ds(c * _E + base, _CH)], idx_s[b], si[b])
        pltpu.async_copy(dst_hbm.at[pl.ds(base, _CH)], idx_d[b], si[b])

    def wait_idx(b):
        pltpu.make_async_copy(dst_hbm.at[pl.ds(0, _CH)], idx_s[b], si[b]).wait()
        pltpu.make_async_copy(dst_hbm.at[pl.ds(0, _CH)], idx_d[b], si[b]).wait()

    def start_gather(b):
        pltpu.async_copy(x2_hbm.at[idx_s[b]], rows[b], sg[b])

    def wait_gather(b):
        pltpu.make_async_copy(x2_hbm.at[idx_s[b]], rows[b], sg[b]).wait()

    load_idx(0, 0)
    load_idx(1, 1)
    wait_idx(0)
    start_gather(0)

    def pair(j, carry):
        for b in range(2):
            k = 2 * j + b

            @pl.when(k + 1 < _NCHUNK)
            def _():
                wait_idx(1 - b)
                start_gather(1 - b)

            wait_gather(b)
            pltpu.sync_copy(rows[b], acc.at[idx_d[b]], add=True)

            @pl.when(k + 2 < _NCHUNK)
            def _():
                load_idx(k + 2, b)

        return carry

    lax.fori_loop(0, _NCHUNK // 2, pair, 0)
    if _NCHUNK % 2 == 1:
        b_last = (_NCHUNK - 1) % 2
        wait_gather(b_last)
        pltpu.sync_copy(rows[b_last], acc.at[idx_d[b_last]], add=True)
    plsc.subcore_barrier()
    for j in range(_ZROUNDS):
        k = s + j * _NS

        @pl.when(k < _ZCHUNKS)
        def _():
            pltpu.sync_copy(
                acc.at[pl.ds(k * _ZROW, _ZROW)],
                out_hbm.at[pl.ds(c * _N + k * _ZROW, _ZROW)],
            )


@functools.partial(
    pl.kernel,
    out_type=jax.ShapeDtypeStruct((_NC * _G, _H), jnp.float32),
    mesh=_mesh,
    compiler_params=_sc_params,
    scratch_types=[
        pltpu.VMEM_SHARED((_G, _H), jnp.float32),
        pltpu.VMEM((8, _H), jnp.float32),
        pltpu.VMEM((_POOL_CH,), jnp.int32),
        pltpu.VMEM((_POOL_CH, _H), jnp.float32),
    ],
)
def _sc_pool(h_hbm, batch_hbm, out_hbm, acc, zbuf, idx_b, rows):
    c = lax.axis_index("c")
    s = lax.axis_index("s")
    for i in range(8):
        zbuf[i, pl.ds(0, 16)] = jnp.zeros((16,), jnp.float32)
        zbuf[i, pl.ds(16, 16)] = jnp.zeros((16,), jnp.float32)
    pltpu.sync_copy(zbuf, acc.at[pl.ds(s * 8, 8)])
    plsc.subcore_barrier()

    def chunk(j, carry):
        k = s + j * _NS

        @pl.when(k < _POOL_CHUNKS_TOTAL)
        def _():
            base = c * _HALF + k * _POOL_CH
            pltpu.sync_copy(batch_hbm.at[pl.ds(base, _POOL_CH)], idx_b)
            pltpu.sync_copy(h_hbm.at[pl.ds(base, _POOL_CH)], rows)
            pltpu.sync_copy(rows, acc.at[idx_b], add=True)

        return carry

    lax.fori_loop(0, _POOL_ROUNDS, chunk, 0)
    plsc.subcore_barrier()
    pltpu.sync_copy(acc.at[pl.ds(s * 8, 8)], out_hbm.at[pl.ds(c * _G + s * 8, 8)])


def _mlp_body(a_ref, x_ref, w1_ref, b1_ref, g_ref, be_ref, w2_ref, b2_ref,
              c_ref, ct_ref, o_ref):
    t = a_ref[...] + x_ref[...]
    h = jnp.dot(t, w1_ref[...], preferred_element_type=jnp.float32) + b1_ref[...]
    hi = lax.Precision.HIGHEST
    ssum = jnp.sum(h, axis=0, keepdims=True)
    m = jnp.dot(jnp.dot(ssum, c_ref[...], precision=hi) * (1.0 / _N),
                ct_ref[...], precision=hi)
    d = h - m
    vsum = jnp.sum(d * d, axis=0, keepdims=True)
    v = jnp.dot(jnp.dot(vsum, c_ref[...], precision=hi) * (1.0 / _N),
                ct_ref[...], precision=hi)
    hn = d / jnp.sqrt(v + 1e-5) * g_ref[...] + be_ref[...]
    h2 = (jnp.dot(jnp.maximum(hn, 0.0), w2_ref[...],
                  preferred_element_type=jnp.float32) + b2_ref[...])
    o_ref[...] = jnp.maximum(h2, 0.0)


_tc_mlp = pl.pallas_call(
    _mlp_body,
    out_shape=jax.ShapeDtypeStruct((_N // 4, 4 * _H), jnp.float32),
)


def _final_body(p_ref, wc_ref, bc_ref, o_ref):
    p = p_ref[0] + p_ref[1]
    o_ref[...] = (jnp.dot(p, wc_ref[...], preferred_element_type=jnp.float32)
                  + bc_ref[0, 0])


_tc_final = pl.pallas_call(
    _final_body,
    out_shape=jax.ShapeDtypeStruct((_G, 1), jnp.float32),
)


def kernel(x, edge_index, batch, W1_0, b1_0, g_0, be_0, W2_0, b2_0,
           W1_r, b1_r, g_r, be_r, W2_r, b2_r, Wc, bc):
    src = edge_index[0]
    dst = edge_index[1]
    xp = jnp.pad(x, ((0, 0), (0, _H - x.shape[1])))
    srcm = jnp.concatenate([2 * src, 2 * src + 1])

    eye4 = jnp.eye(4, dtype=jnp.float32)
    coll = jnp.tile(jnp.eye(_H, dtype=jnp.float32), (4, 1))
    collT = coll.T

    W1_0p = jnp.pad(W1_0, ((0, _H - W1_0.shape[0]), (0, 0)))
    w1s = [W1_0p] + [W1_r[i] for i in range(_L - 1)]
    b1s = [b1_0] + [b1_r[i] for i in range(_L - 1)]
    gs = [g_0] + [g_r[i] for i in range(_L - 1)]
    bes = [be_0] + [be_r[i] for i in range(_L - 1)]
    w2s = [W2_0] + [W2_r[i] for i in range(_L - 1)]
    b2s = [b2_0] + [b2_r[i] for i in range(_L - 1)]

    h = xp
    for i in range(_L):
        agg2 = _sc_agg(h.reshape(_NC * _N, _HH), srcm, dst)
        agg = jnp.concatenate(
            [agg2[:_N], agg2[_N:]], axis=1)
        h4 = _tc_mlp(
            agg.reshape(_N // 4, 4 * _H),
            h.reshape(_N // 4, 4 * _H),
            jnp.kron(eye4, w1s[i]),
            jnp.tile(b1s[i], 4).reshape(1, 4 * _H),
            jnp.tile(gs[i], 4).reshape(1, 4 * _H),
            jnp.tile(bes[i], 4).reshape(1, 4 * _H),
            jnp.kron(eye4, w2s[i]),
            jnp.tile(b2s[i], 4).reshape(1, 4 * _H),
            coll,
            collT,
        )
        h = h4.reshape(_N, _H)

    pooled = _sc_pool(h, batch)
    out = _tc_final(pooled.reshape(_NC, _G, _H), Wc, bc.reshape(1, 1))
    return out.reshape(-1)
